# Initial kernel scaffold; baseline (speedup 1.0000x reference)
import functools

import jax
import jax.numpy as jnp
from jax import lax
from jax.experimental import pallas as pl
from jax.experimental.pallas import tpu as pltpu
from jax.experimental.pallas import tpu_sc as plsc

D = 64
H = 128
NC, NS = 2, 16
NW = NC * NS
K = 128


def _sc_gather_add(ids, cats, item_table, cat_table):
    n_rows = ids.shape[0]
    rows_per_w = n_rows // NW
    n_chunks = rows_per_w // K
    mesh = plsc.VectorSubcoreMesh(
        core_axis_name="c", subcore_axis_name="s", num_cores=NC, num_subcores=NS
    )

    @functools.partial(
        pl.kernel,
        out_type=jax.ShapeDtypeStruct((n_rows, D), jnp.float32),
        mesh=mesh,
        scratch_types=[
            pltpu.VMEM((K,), jnp.int32),
            pltpu.VMEM((K,), jnp.int32),
            pltpu.VMEM((K, D), jnp.float32),
            pltpu.VMEM((K, D), jnp.float32),
            pltpu.SemaphoreType.DMA,
            pltpu.SemaphoreType.DMA,
        ],
    )
    def sc_kernel(ids_hbm, cats_hbm, itab_hbm, ctab_hbm, out_hbm,
                  idx_i, idx_c, ebuf, cbuf, sem_i, sem_c):
        wid = lax.axis_index("s") * NC + lax.axis_index("c")
        base = wid * rows_per_w

        def chunk_body(i, carry):
            off = base + i * K
            pltpu.sync_copy(ids_hbm.at[pl.ds(off, K)], idx_i)
            pltpu.sync_copy(cats_hbm.at[pl.ds(off, K)], idx_c)
            cp_i = pltpu.async_copy(itab_hbm.at[idx_i], ebuf, sem_i)
            cp_c = pltpu.async_copy(ctab_hbm.at[idx_c], cbuf, sem_c)
            cp_i.wait()
            cp_c.wait()

            def add_row(j, c2):
                for t in range(D // 16):
                    sl = pl.ds(t * 16, 16)
                    ebuf[j, sl] = ebuf[j, sl] + cbuf[j, sl]
                return c2

            lax.fori_loop(0, K, add_row, 0)
            pltpu.sync_copy(ebuf, out_hbm.at[pl.ds(off, K)])
            return carry

        lax.fori_loop(0, n_chunks, chunk_body, 0)

    return sc_kernel(ids, cats, item_table, cat_table)


def _tc_mlp_normalize(e, W1, b1, W2, b2):
    n_rows = e.shape[0]
    blk = 2048
    grid = n_rows // blk

    def body(e_ref, w1_ref, b1_ref, w2_ref, b2_ref, o_ref):
        ev = e_ref[...]
        h = jnp.dot(ev, w1_ref[...], preferred_element_type=jnp.float32)
        h = jnp.maximum(h + b1_ref[...], 0.0)
        r = jnp.dot(h, w2_ref[...], preferred_element_type=jnp.float32)
        r = r + b2_ref[...] + ev
        norm = jnp.sqrt(jnp.sum(r * r, axis=-1, keepdims=True))
        o_ref[...] = r / jnp.maximum(norm, 1e-6)

    return pl.pallas_call(
        body,
        grid=(grid,),
        in_specs=[
            pl.BlockSpec((blk, D), lambda i: (i, 0)),
            pl.BlockSpec((D, H), lambda i: (0, 0)),
            pl.BlockSpec((1, H), lambda i: (0, 0)),
            pl.BlockSpec((H, D), lambda i: (0, 0)),
            pl.BlockSpec((1, D), lambda i: (0, 0)),
        ],
        out_specs=pl.BlockSpec((blk, D), lambda i: (i, 0)),
        out_shape=jax.ShapeDtypeStruct((n_rows, D), jnp.float32),
    )(e, W1, b1, W2, b2)


def kernel(item_ids, categories, item_table, cat_table, W1, b1, W2, b2):
    B, C = item_ids.shape
    n_rows = B * C
    ids = item_ids.reshape(n_rows).astype(jnp.int32)
    cats = categories.reshape(n_rows).astype(jnp.int32)
    e = _sc_gather_add(ids, cats, item_table, cat_table)
    out = _tc_mlp_normalize(e, W1, b1.reshape(1, H), W2, b2.reshape(1, D))
    return out.reshape(B, C, D)


# speedup vs baseline: 1.7148x; 1.7148x over previous
import functools

import jax
import jax.numpy as jnp
from jax import lax
from jax.experimental import pallas as pl
from jax.experimental.pallas import tpu as pltpu
from jax.experimental.pallas import tpu_sc as plsc

D = 64
H = 128
NC, NS = 2, 16
NW = NC * NS
K = 128


def _sc_gather_add(ids, cats, item_table, cat_table):
    n_rows = ids.shape[0]
    rows_per_w = n_rows // NW
    n_chunks = rows_per_w // K
    mesh = plsc.VectorSubcoreMesh(
        core_axis_name="c", subcore_axis_name="s", num_cores=NC, num_subcores=NS
    )

    @functools.partial(
        pl.kernel,
        out_type=jax.ShapeDtypeStruct((n_rows, D), jnp.float32),
        mesh=mesh,
        scratch_types=[
            pltpu.VMEM((K,), jnp.int32),
            pltpu.VMEM((K,), jnp.int32),
            pltpu.VMEM((K, D), jnp.float32),
            pltpu.VMEM((K, D), jnp.float32),
            pltpu.SemaphoreType.DMA,
            pltpu.SemaphoreType.DMA,
        ],
        compiler_params=pltpu.CompilerParams(use_tc_tiling_on_sc=False),
    )
    def sc_kernel(ids_hbm, cats_hbm, itab_hbm, ctab_hbm, out_hbm,
                  idx_i, idx_c, ebuf, cbuf, sem_i, sem_c):
        wid = lax.axis_index("s") * NC + lax.axis_index("c")
        base = wid * rows_per_w

        def chunk_body(i, carry):
            off = base + i * K
            pltpu.sync_copy(ids_hbm.at[pl.ds(off, K)], idx_i)
            pltpu.sync_copy(cats_hbm.at[pl.ds(off, K)], idx_c)
            cp_i = pltpu.async_copy(itab_hbm.at[idx_i], ebuf, sem_i)
            cp_c = pltpu.async_copy(ctab_hbm.at[idx_c], cbuf, sem_c)
            cp_i.wait()
            cp_c.wait()

            def add_row(j, c2):
                for t in range(D // 16):
                    sl = pl.ds(t * 16, 16)
                    ebuf[j, sl] = ebuf[j, sl] + cbuf[j, sl]
                return c2

            lax.fori_loop(0, K, add_row, 0)
            pltpu.sync_copy(ebuf, out_hbm.at[pl.ds(off, K)])
            return carry

        lax.fori_loop(0, n_chunks, chunk_body, 0)

    return sc_kernel(ids, cats, item_table, cat_table)


def _tc_mlp_normalize(e, W1, b1, W2, b2):
    n_rows = e.shape[0]
    blk = 2048
    grid = n_rows // blk

    def body(e_ref, w1_ref, b1_ref, w2_ref, b2_ref, o_ref):
        ev = e_ref[...]
        h = jnp.dot(ev, w1_ref[...], preferred_element_type=jnp.float32)
        h = jnp.maximum(h + b1_ref[...], 0.0)
        r = jnp.dot(h, w2_ref[...], preferred_element_type=jnp.float32)
        r = r + b2_ref[...] + ev
        norm = jnp.sqrt(jnp.sum(r * r, axis=-1, keepdims=True))
        o_ref[...] = r / jnp.maximum(norm, 1e-6)

    return pl.pallas_call(
        body,
        grid=(grid,),
        in_specs=[
            pl.BlockSpec((blk, D), lambda i: (i, 0)),
            pl.BlockSpec((D, H), lambda i: (0, 0)),
            pl.BlockSpec((1, H), lambda i: (0, 0)),
            pl.BlockSpec((H, D), lambda i: (0, 0)),
            pl.BlockSpec((1, D), lambda i: (0, 0)),
        ],
        out_specs=pl.BlockSpec((blk, D), lambda i: (i, 0)),
        out_shape=jax.ShapeDtypeStruct((n_rows, D), jnp.float32),
    )(e, W1, b1, W2, b2)


def kernel(item_ids, categories, item_table, cat_table, W1, b1, W2, b2):
    B, C = item_ids.shape
    n_rows = B * C
    ids = item_ids.reshape(n_rows).astype(jnp.int32)
    cats = categories.reshape(n_rows).astype(jnp.int32)
    e = _sc_gather_add(ids, cats, item_table, cat_table)
    out = _tc_mlp_normalize(e, W1, b1.reshape(1, H), W2, b2.reshape(1, D))
    return out.reshape(B, C, D)


# speedup vs baseline: 2.1013x; 1.2254x over previous
import functools

import jax
import jax.numpy as jnp
from jax import lax
from jax.experimental import pallas as pl
from jax.experimental.pallas import tpu as pltpu
from jax.experimental.pallas import tpu_sc as plsc

D = 64
H = 128
NC, NS = 2, 16
NW = NC * NS
K = 128


def _sc_gather_add(ids, cats, item_table, cat_table):
    n_rows = ids.shape[0]
    rows_per_w = n_rows // NW
    n_chunks = rows_per_w // K
    mesh = plsc.VectorSubcoreMesh(
        core_axis_name="c", subcore_axis_name="s", num_cores=NC, num_subcores=NS
    )

    @functools.partial(
        pl.kernel,
        out_type=jax.ShapeDtypeStruct((n_rows, D), jnp.float32),
        mesh=mesh,
        scratch_types=[
            pltpu.VMEM((K,), jnp.int32),
            pltpu.VMEM((K,), jnp.int32),
            pltpu.VMEM((K, D), jnp.float32),
            pltpu.VMEM((K, D), jnp.float32),
            pltpu.SemaphoreType.DMA,
            pltpu.SemaphoreType.DMA,
        ],
        compiler_params=pltpu.CompilerParams(use_tc_tiling_on_sc=False),
    )
    def sc_kernel(ids_hbm, cats_hbm, itab_hbm, ctab_hbm, out_hbm,
                  idx_i, idx_c, ebuf, cbuf, sem_i, sem_c):
        wid = lax.axis_index("s") * NC + lax.axis_index("c")
        base = wid * rows_per_w

        def chunk_body(i, carry):
            off = base + i * K
            pltpu.sync_copy(ids_hbm.at[pl.ds(off, K)], idx_i)
            pltpu.sync_copy(cats_hbm.at[pl.ds(off, K)], idx_c)
            cp_i = pltpu.async_copy(itab_hbm.at[idx_i], ebuf, sem_i)
            cp_c = pltpu.async_copy(ctab_hbm.at[idx_c], cbuf, sem_c)
            cp_i.wait()
            cp_c.wait()

            def add_row(j, c2):
                for t in range(D // 16):
                    sl = pl.ds(t * 16, 16)
                    ebuf[j, sl] = ebuf[j, sl] + cbuf[j, sl]
                return c2

            lax.fori_loop(0, K, add_row, 0)
            pltpu.sync_copy(ebuf, out_hbm.at[pl.ds(off, K)])
            return carry

        lax.fori_loop(0, n_chunks, chunk_body, 0)

    return sc_kernel(ids, cats, item_table, cat_table)


def _tc_mlp_normalize(e2, W1d, b1d, W2d, b2d):
    n2 = e2.shape[0]
    blk = 1024
    grid = n2 // blk

    def body(e_ref, w1_ref, b1_ref, w2_ref, b2_ref, o_ref):
        ev = e_ref[...]
        h = jnp.dot(ev, w1_ref[...], preferred_element_type=jnp.float32)
        h = jnp.maximum(h + b1_ref[...], 0.0)
        r = jnp.dot(h, w2_ref[...], preferred_element_type=jnp.float32)
        r = r + b2_ref[...] + ev
        s = r * r
        col = lax.broadcasted_iota(jnp.int32, (1, 2 * D), 1)
        mask_l = (col < D).astype(jnp.float32)
        mask_r = 1.0 - mask_l
        nl = jnp.sum(s * mask_l, axis=-1, keepdims=True)
        nr = jnp.sum(s * mask_r, axis=-1, keepdims=True)
        dl = jnp.maximum(jnp.sqrt(nl), 1e-6)
        dr = jnp.maximum(jnp.sqrt(nr), 1e-6)
        denom = mask_l * dl + mask_r * dr
        o_ref[...] = r / denom

    return pl.pallas_call(
        body,
        grid=(grid,),
        in_specs=[
            pl.BlockSpec((blk, 2 * D), lambda i: (i, 0)),
            pl.BlockSpec((2 * D, 2 * H), lambda i: (0, 0)),
            pl.BlockSpec((1, 2 * H), lambda i: (0, 0)),
            pl.BlockSpec((2 * H, 2 * D), lambda i: (0, 0)),
            pl.BlockSpec((1, 2 * D), lambda i: (0, 0)),
        ],
        out_specs=pl.BlockSpec((blk, 2 * D), lambda i: (i, 0)),
        out_shape=jax.ShapeDtypeStruct((n2, 2 * D), jnp.float32),
    )(e2, W1d, b1d, W2d, b2d)


def kernel(item_ids, categories, item_table, cat_table, W1, b1, W2, b2):
    B, C = item_ids.shape
    n_rows = B * C
    ids = item_ids.reshape(n_rows).astype(jnp.int32)
    cats = categories.reshape(n_rows).astype(jnp.int32)
    e = _sc_gather_add(ids, cats, item_table, cat_table)
    e2 = e.reshape(n_rows // 2, 2 * D)
    W1d = (
        jnp.zeros((2 * D, 2 * H), jnp.float32)
        .at[:D, :H].set(W1)
        .at[D:, H:].set(W1)
    )
    W2d = (
        jnp.zeros((2 * H, 2 * D), jnp.float32)
        .at[:H, :D].set(W2)
        .at[H:, D:].set(W2)
    )
    b1d = jnp.concatenate([b1, b1]).reshape(1, 2 * H)
    b2d = jnp.concatenate([b2, b2]).reshape(1, 2 * D)
    out2 = _tc_mlp_normalize(e2, W1d, b1d, W2d, b2d)
    return out2.reshape(B, C, D)


# speedup vs baseline: 2.4041x; 1.1441x over previous
import functools

import jax
import jax.numpy as jnp
from jax import lax
from jax.experimental import pallas as pl
from jax.experimental.pallas import tpu as pltpu
from jax.experimental.pallas import tpu_sc as plsc

D = 64
H = 128
NC, NS = 2, 16
NW = NC * NS
K = 128


def _sc_gather_add(ids, cats, itab, ctab):
    n_rows = ids.shape[0]
    rows_per_w = n_rows // NW
    n_chunks = rows_per_w // K
    mesh = plsc.VectorSubcoreMesh(
        core_axis_name="c", subcore_axis_name="s", num_cores=NC, num_subcores=NS
    )

    @functools.partial(
        pl.kernel,
        out_type=jax.ShapeDtypeStruct((n_rows, D), jnp.float32),
        mesh=mesh,
        scratch_types=[
            pltpu.VMEM((2, K), jnp.int32),
            pltpu.VMEM((2, K), jnp.int32),
            pltpu.VMEM((2, K, D), jnp.float32),
            pltpu.VMEM((2, K, D), jnp.float32),
            pltpu.VMEM((2, K, D), jnp.float32),
            pltpu.SemaphoreType.DMA,
            pltpu.SemaphoreType.DMA,
            pltpu.SemaphoreType.DMA,
            pltpu.SemaphoreType.DMA,
            pltpu.SemaphoreType.DMA,
            pltpu.SemaphoreType.DMA,
        ],
        compiler_params=pltpu.CompilerParams(use_tc_tiling_on_sc=False),
    )
    def sc_kernel(ids_hbm, cats_hbm, itab_hbm, ctab_hbm, out_hbm,
                  idx_i, idx_c, ebuf, cbuf, obuf,
                  sgi0, sgc0, so0, sgi1, sgc1, so1):
        itab = itab_hbm
        ctab = ctab_hbm
        sgi = (sgi0, sgi1)
        sgc = (sgc0, sgc1)
        so = (so0, so1)
        wid = lax.axis_index("s") * NC + lax.axis_index("c")
        base = wid * rows_per_w

        def stage_idx(c, s):
            off = base + c * K
            pltpu.sync_copy(ids_hbm.at[pl.ds(off, K)], idx_i.at[s])
            pltpu.sync_copy(cats_hbm.at[pl.ds(off, K)], idx_c.at[s])

        def issue_gather(s):
            pltpu.async_copy(itab.at[idx_i.at[s]], ebuf.at[s], sgi[s])
            pltpu.async_copy(ctab.at[idx_c.at[s]], cbuf.at[s], sgc[s])

        def wait_gather(s):
            pltpu.make_async_copy(itab.at[idx_i.at[s]], ebuf.at[s], sgi[s]).wait()
            pltpu.make_async_copy(ctab.at[idx_c.at[s]], cbuf.at[s], sgc[s]).wait()

        def do_add(s):
            def add_row(j, carry):
                for t in range(D // 16):
                    sl = pl.ds(t * 16, 16)
                    obuf[s, j, sl] = ebuf[s, j, sl] + cbuf[s, j, sl]
                return carry
            lax.fori_loop(0, K, add_row, 0)

        def issue_out(c, s):
            off = base + c * K
            pltpu.async_copy(obuf.at[s], out_hbm.at[pl.ds(off, K)], so[s])

        def wait_out(s):
            pltpu.make_async_copy(
                obuf.at[s], out_hbm.at[pl.ds(base, K)], so[s]
            ).wait()

        stage_idx(0, 0)
        issue_gather(0)
        stage_idx(1, 1)
        issue_gather(1)

        def body(g2, carry):
            for s in (0, 1):
                c = 2 * g2 + s
                wait_gather(s)

                @pl.when(g2 > 0)
                def _():
                    wait_out(s)

                do_add(s)
                issue_out(c, s)

                @pl.when(c + 2 < n_chunks)
                def _():
                    stage_idx(c + 2, s)
                    issue_gather(s)
            return carry

        lax.fori_loop(0, n_chunks // 2, body, 0)
        wait_out(0)
        wait_out(1)

    return sc_kernel(ids, cats, itab, ctab)


def _tc_mlp_normalize(e2, W1d, b1d, W2d, b2d):
    n2 = e2.shape[0]
    blk = 1024
    grid = n2 // blk

    def body(e_ref, w1_ref, b1_ref, w2_ref, b2_ref, o_ref):
        ev = e_ref[...]
        h = jnp.dot(ev, w1_ref[...], preferred_element_type=jnp.float32)
        h = jnp.maximum(h + b1_ref[...], 0.0)
        r = jnp.dot(h, w2_ref[...], preferred_element_type=jnp.float32)
        r = r + b2_ref[...] + ev
        s = r * r
        col = lax.broadcasted_iota(jnp.int32, (1, 2 * D), 1)
        mask_l = (col < D).astype(jnp.float32)
        mask_r = 1.0 - mask_l
        nl = jnp.sum(s * mask_l, axis=-1, keepdims=True)
        nr = jnp.sum(s * mask_r, axis=-1, keepdims=True)
        dl = jnp.maximum(jnp.sqrt(nl), 1e-6)
        dr = jnp.maximum(jnp.sqrt(nr), 1e-6)
        denom = mask_l * dl + mask_r * dr
        o_ref[...] = r / denom

    return pl.pallas_call(
        body,
        grid=(grid,),
        in_specs=[
            pl.BlockSpec((blk, 2 * D), lambda i: (i, 0)),
            pl.BlockSpec((2 * D, 2 * H), lambda i: (0, 0)),
            pl.BlockSpec((1, 2 * H), lambda i: (0, 0)),
            pl.BlockSpec((2 * H, 2 * D), lambda i: (0, 0)),
            pl.BlockSpec((1, 2 * D), lambda i: (0, 0)),
        ],
        out_specs=pl.BlockSpec((blk, 2 * D), lambda i: (i, 0)),
        out_shape=jax.ShapeDtypeStruct((n2, 2 * D), jnp.float32),
    )(e2, W1d, b1d, W2d, b2d)


def kernel(item_ids, categories, item_table, cat_table, W1, b1, W2, b2):
    B, C = item_ids.shape
    n_rows = B * C
    ids = item_ids.reshape(n_rows).astype(jnp.int32)
    cats = categories.reshape(n_rows).astype(jnp.int32)
    e = _sc_gather_add(ids, cats, item_table, cat_table)
    e2 = e.reshape(n_rows // 2, 2 * D)
    W1d = (
        jnp.zeros((2 * D, 2 * H), jnp.float32)
        .at[:D, :H].set(W1)
        .at[D:, H:].set(W1)
    )
    W2d = (
        jnp.zeros((2 * H, 2 * D), jnp.float32)
        .at[:H, :D].set(W2)
        .at[H:, D:].set(W2)
    )
    b1d = jnp.concatenate([b1, b1]).reshape(1, 2 * H)
    b2d = jnp.concatenate([b2, b2]).reshape(1, 2 * D)
    out2 = _tc_mlp_normalize(e2, W1d, b1d, W2d, b2d)
    return out2.reshape(B, C, D)
